# Initial kernel scaffold; baseline (speedup 1.0000x reference)
import jax
import jax.numpy as jnp
from jax.experimental import pallas as pl


def kernel(x, hidden_states, masks, W_ih_0, W_hh_0, b_ih_0, b_hh_0, W_ih_1, W_hh_1, b_ih_1, b_hh_1):
    raise NotImplementedError("write your pallas kernel here")


# speedup vs baseline: 1.6152x; 1.6152x over previous
import functools

import jax
import jax.numpy as jnp
from jax.experimental import pallas as pl
from jax.experimental.pallas import tpu as pltpu

N = 4096
H = 512
G = 4 * H


def _sigmoid(x):
    return 0.5 * (jnp.tanh(0.5 * x) + 1.0)


def _lstm_kernel(x_ref, hs_ref, m_ref, wi0_ref, wh0_ref, wi1_ref, wh1_ref,
                 b0_ref, b1_ref, out_ref, hout_ref):
    m = m_ref[...]
    hs = hs_ref[...] * m
    h0 = hs[:, 0 * H:1 * H]
    h1 = hs[:, 1 * H:2 * H]
    c0 = hs[:, 2 * H:3 * H]
    c1 = hs[:, 3 * H:4 * H]

    xb = x_ref[...].astype(jnp.bfloat16)
    gates0 = (
        jnp.dot(xb, wi0_ref[...], preferred_element_type=jnp.float32)
        + jnp.dot(h0.astype(jnp.bfloat16), wh0_ref[...],
                  preferred_element_type=jnp.float32)
        + b0_ref[...]
    )
    i0 = _sigmoid(gates0[:, 0 * H:1 * H])
    f0 = _sigmoid(gates0[:, 1 * H:2 * H])
    g0 = jnp.tanh(gates0[:, 2 * H:3 * H])
    o0 = _sigmoid(gates0[:, 3 * H:4 * H])
    c0n = f0 * c0 + i0 * g0
    h0n = o0 * jnp.tanh(c0n)

    gates1 = (
        jnp.dot(h0n.astype(jnp.bfloat16), wi1_ref[...],
                preferred_element_type=jnp.float32)
        + jnp.dot(h1.astype(jnp.bfloat16), wh1_ref[...],
                  preferred_element_type=jnp.float32)
        + b1_ref[...]
    )
    i1 = _sigmoid(gates1[:, 0 * H:1 * H])
    f1 = _sigmoid(gates1[:, 1 * H:2 * H])
    g1 = jnp.tanh(gates1[:, 2 * H:3 * H])
    o1 = _sigmoid(gates1[:, 3 * H:4 * H])
    c1n = f1 * c1 + i1 * g1
    h1n = o1 * jnp.tanh(c1n)

    out_ref[...] = h1n
    hout_ref[:, 0 * H:1 * H] = h0n
    hout_ref[:, 1 * H:2 * H] = h1n
    hout_ref[:, 2 * H:3 * H] = c0n
    hout_ref[:, 3 * H:4 * H] = c1n


@functools.partial(jax.jit, static_argnames=("bn", "interpret"))
def _run(x, hs2, mf, wi0, wh0, wi1, wh1, b0, b1, bn=512, interpret=False):
    grid = (N // bn,)
    row = lambda i: (i, 0)
    rep = lambda i: (0, 0)
    out, hout = pl.pallas_call(
        _lstm_kernel,
        grid=grid,
        in_specs=[
            pl.BlockSpec((bn, H), row),
            pl.BlockSpec((bn, 4 * H), row),
            pl.BlockSpec((bn, 1), row),
            pl.BlockSpec((H, G), rep),
            pl.BlockSpec((H, G), rep),
            pl.BlockSpec((H, G), rep),
            pl.BlockSpec((H, G), rep),
            pl.BlockSpec((1, G), rep),
            pl.BlockSpec((1, G), rep),
        ],
        out_specs=[
            pl.BlockSpec((bn, H), row),
            pl.BlockSpec((bn, 4 * H), row),
        ],
        out_shape=[
            jax.ShapeDtypeStruct((N, H), jnp.float32),
            jax.ShapeDtypeStruct((N, 4 * H), jnp.float32),
        ],
        compiler_params=pltpu.CompilerParams(
            dimension_semantics=("arbitrary",),
        ),
        interpret=interpret,
    )(x, hs2, mf, wi0, wh0, wi1, wh1, b0, b1)
    return out, hout


def kernel(x, hidden_states, masks, W_ih_0, W_hh_0, b_ih_0, b_hh_0,
           W_ih_1, W_hh_1, b_ih_1, b_hh_1, *, interpret=False):
    mf = masks.astype(jnp.float32)
    hs2 = hidden_states.reshape(N, 4 * H)
    wi0 = W_ih_0.astype(jnp.bfloat16).T
    wh0 = W_hh_0.astype(jnp.bfloat16).T
    wi1 = W_ih_1.astype(jnp.bfloat16).T
    wh1 = W_hh_1.astype(jnp.bfloat16).T
    b0 = (b_ih_0 + b_hh_0).reshape(1, G)
    b1 = (b_ih_1 + b_hh_1).reshape(1, G)
    out, hout = _run(x, hs2, mf, wi0, wh0, wi1, wh1, b0, b1,
                     interpret=interpret)
    return out, hout.reshape(N, 2 * 2, H)


# speedup vs baseline: 1.6316x; 1.0101x over previous
import functools

import jax
import jax.numpy as jnp
from jax.experimental import pallas as pl
from jax.experimental.pallas import tpu as pltpu

N = 4096
H = 512
G = 4 * H


def _sigmoid(x):
    return 0.5 * (jnp.tanh(0.5 * x) + 1.0)


def _dot_t(a, b):
    return jax.lax.dot_general(
        a, b, dimension_numbers=(((1,), (1,)), ((), ())),
        preferred_element_type=jnp.float32)


def _lstm_kernel(x_ref, hs_ref, m_ref, wi0_ref, wh0_ref, wi1_ref, wh1_ref,
                 b0_ref, b1_ref, out_ref, hout_ref):
    m = m_ref[...]
    hs = hs_ref[...] * m
    h0 = hs[:, 0 * H:1 * H]
    h1 = hs[:, 1 * H:2 * H]
    c0 = hs[:, 2 * H:3 * H]
    c1 = hs[:, 3 * H:4 * H]

    xb = x_ref[...].astype(jnp.bfloat16)
    gates0 = (
        _dot_t(xb, wi0_ref[...].astype(jnp.bfloat16))
        + _dot_t(h0.astype(jnp.bfloat16), wh0_ref[...].astype(jnp.bfloat16))
        + b0_ref[...]
    )
    i0 = _sigmoid(gates0[:, 0 * H:1 * H])
    f0 = _sigmoid(gates0[:, 1 * H:2 * H])
    g0 = jnp.tanh(gates0[:, 2 * H:3 * H])
    o0 = _sigmoid(gates0[:, 3 * H:4 * H])
    c0n = f0 * c0 + i0 * g0
    h0n = o0 * jnp.tanh(c0n)

    gates1 = (
        _dot_t(h0n.astype(jnp.bfloat16), wi1_ref[...].astype(jnp.bfloat16))
        + _dot_t(h1.astype(jnp.bfloat16), wh1_ref[...].astype(jnp.bfloat16))
        + b1_ref[...]
    )
    i1 = _sigmoid(gates1[:, 0 * H:1 * H])
    f1 = _sigmoid(gates1[:, 1 * H:2 * H])
    g1 = jnp.tanh(gates1[:, 2 * H:3 * H])
    o1 = _sigmoid(gates1[:, 3 * H:4 * H])
    c1n = f1 * c1 + i1 * g1
    h1n = o1 * jnp.tanh(c1n)

    out_ref[...] = h1n
    hout_ref[:, 0 * H:1 * H] = h0n
    hout_ref[:, 1 * H:2 * H] = h1n
    hout_ref[:, 2 * H:3 * H] = c0n
    hout_ref[:, 3 * H:4 * H] = c1n


@functools.partial(jax.jit, static_argnames=("bn", "interpret"))
def _run(x, hs2, mf, wi0, wh0, wi1, wh1, b0, b1, bn=512, interpret=False):
    grid = (N // bn,)
    row = lambda i: (i, 0)
    rep = lambda i: (0, 0)
    out, hout = pl.pallas_call(
        _lstm_kernel,
        grid=grid,
        in_specs=[
            pl.BlockSpec((bn, H), row),
            pl.BlockSpec((bn, 4 * H), row),
            pl.BlockSpec((bn, 1), row),
            pl.BlockSpec((G, H), rep),
            pl.BlockSpec((G, H), rep),
            pl.BlockSpec((G, H), rep),
            pl.BlockSpec((G, H), rep),
            pl.BlockSpec((1, G), rep),
            pl.BlockSpec((1, G), rep),
        ],
        out_specs=[
            pl.BlockSpec((bn, H), row),
            pl.BlockSpec((bn, 4 * H), row),
        ],
        out_shape=[
            jax.ShapeDtypeStruct((N, H), jnp.float32),
            jax.ShapeDtypeStruct((N, 4 * H), jnp.float32),
        ],
        compiler_params=pltpu.CompilerParams(
            dimension_semantics=("arbitrary",),
        ),
        interpret=interpret,
    )(x, hs2, mf, wi0, wh0, wi1, wh1, b0, b1)
    return out, hout


def kernel(x, hidden_states, masks, W_ih_0, W_hh_0, b_ih_0, b_hh_0,
           W_ih_1, W_hh_1, b_ih_1, b_hh_1, *, interpret=False):
    mf = masks.astype(jnp.float32)
    hs2 = hidden_states.reshape(N, 4 * H)
    b0 = (b_ih_0 + b_hh_0).reshape(1, G)
    b1 = (b_ih_1 + b_hh_1).reshape(1, G)
    out, hout = _run(x, hs2, mf, W_ih_0, W_hh_0, W_ih_1, W_hh_1, b0, b1,
                     interpret=interpret)
    return out, hout.reshape(N, 2 * 2, H)


# speedup vs baseline: 3.4555x; 2.1179x over previous
import functools

import jax
import jax.numpy as jnp
from jax.experimental import pallas as pl
from jax.experimental.pallas import tpu as pltpu

N = 4096
H = 512
G = 4 * H
BN = 512


def _sigmoid(x):
    return 0.5 * (jnp.tanh(0.5 * x) + 1.0)


def _dot_t(a, b):
    return jax.lax.dot_general(
        a, b, dimension_numbers=(((1,), (1,)), ((), ())),
        preferred_element_type=jnp.float32)


def _hid_in_copy(hid_hbm, hin_buf, in_sems, step, slot, j):
    return pltpu.make_async_copy(
        hid_hbm.at[pl.ds(step * BN, BN), j],
        hin_buf.at[slot, j],
        in_sems.at[slot, j])


def _hid_out_copy(hout_hbm, hout_buf, out_sems, step, slot, j):
    return pltpu.make_async_copy(
        hout_buf.at[slot, j],
        hout_hbm.at[pl.ds(step * BN, BN), j],
        out_sems.at[slot, j])


def _lstm_kernel(x_ref, m_ref, wi0_ref, wh0_ref, wi1_ref, wh1_ref,
                 b0_ref, b1_ref, hid_hbm, out_ref, hout_hbm,
                 hin_buf, hout_buf, in_sems, out_sems):
    i = pl.program_id(0)
    nsteps = pl.num_programs(0)
    slot = jax.lax.rem(i, 2)
    nslot = jax.lax.rem(i + 1, 2)

    @pl.when(i == 0)
    def _():
        for j in range(4):
            _hid_in_copy(hid_hbm, hin_buf, in_sems, 0, 0, j).start()

    @pl.when(i + 1 < nsteps)
    def _():
        for j in range(4):
            _hid_in_copy(hid_hbm, hin_buf, in_sems, i + 1, nslot, j).start()

    for j in range(4):
        _hid_in_copy(hid_hbm, hin_buf, in_sems, i, slot, j).wait()

    m = m_ref[...]
    h0 = hin_buf[slot, 0] * m
    h1 = hin_buf[slot, 1] * m
    c0 = hin_buf[slot, 2] * m
    c1 = hin_buf[slot, 3] * m

    xb = x_ref[...].astype(jnp.bfloat16)
    gates0 = (
        _dot_t(xb, wi0_ref[...].astype(jnp.bfloat16))
        + _dot_t(h0.astype(jnp.bfloat16), wh0_ref[...].astype(jnp.bfloat16))
        + b0_ref[...]
    )
    i0 = _sigmoid(gates0[:, 0 * H:1 * H])
    f0 = _sigmoid(gates0[:, 1 * H:2 * H])
    g0 = jnp.tanh(gates0[:, 2 * H:3 * H])
    o0 = _sigmoid(gates0[:, 3 * H:4 * H])
    c0n = f0 * c0 + i0 * g0
    h0n = o0 * jnp.tanh(c0n)

    gates1 = (
        _dot_t(h0n.astype(jnp.bfloat16), wi1_ref[...].astype(jnp.bfloat16))
        + _dot_t(h1.astype(jnp.bfloat16), wh1_ref[...].astype(jnp.bfloat16))
        + b1_ref[...]
    )
    i1 = _sigmoid(gates1[:, 0 * H:1 * H])
    f1 = _sigmoid(gates1[:, 1 * H:2 * H])
    g1 = jnp.tanh(gates1[:, 2 * H:3 * H])
    o1 = _sigmoid(gates1[:, 3 * H:4 * H])
    c1n = f1 * c1 + i1 * g1
    h1n = o1 * jnp.tanh(c1n)

    out_ref[...] = h1n

    @pl.when(i >= 2)
    def _():
        for j in range(4):
            _hid_out_copy(hout_hbm, hout_buf, out_sems, i - 2, slot, j).wait()

    hout_buf[slot, 0] = h0n
    hout_buf[slot, 1] = h1n
    hout_buf[slot, 2] = c0n
    hout_buf[slot, 3] = c1n
    for j in range(4):
        _hid_out_copy(hout_hbm, hout_buf, out_sems, i, slot, j).start()

    @pl.when(i == nsteps - 1)
    def _():
        for j in range(4):
            _hid_out_copy(hout_hbm, hout_buf, out_sems, i - 1, nslot, j).wait()
            _hid_out_copy(hout_hbm, hout_buf, out_sems, i, slot, j).wait()


@functools.partial(jax.jit, static_argnames=("interpret",))
def _run(x, hs, mf, wi0, wh0, wi1, wh1, b0, b1, interpret=False):
    grid = (N // BN,)
    row = lambda i: (i, 0)
    rep = lambda i: (0, 0)
    out, hout = pl.pallas_call(
        _lstm_kernel,
        grid=grid,
        in_specs=[
            pl.BlockSpec((BN, H), row),
            pl.BlockSpec((BN, 1), row),
            pl.BlockSpec((G, H), rep),
            pl.BlockSpec((G, H), rep),
            pl.BlockSpec((G, H), rep),
            pl.BlockSpec((G, H), rep),
            pl.BlockSpec((1, G), rep),
            pl.BlockSpec((1, G), rep),
            pl.BlockSpec(memory_space=pltpu.MemorySpace.HBM),
        ],
        out_specs=[
            pl.BlockSpec((BN, H), row),
            pl.BlockSpec(memory_space=pltpu.MemorySpace.HBM),
        ],
        out_shape=[
            jax.ShapeDtypeStruct((N, H), jnp.float32),
            jax.ShapeDtypeStruct((N, 4, H), jnp.float32),
        ],
        scratch_shapes=[
            pltpu.VMEM((2, 4, BN, H), jnp.float32),
            pltpu.VMEM((2, 4, BN, H), jnp.float32),
            pltpu.SemaphoreType.DMA((2, 4)),
            pltpu.SemaphoreType.DMA((2, 4)),
        ],
        compiler_params=pltpu.CompilerParams(
            dimension_semantics=("arbitrary",),
        ),
        interpret=interpret,
    )(x, mf, wi0, wh0, wi1, wh1, b0, b1, hs)
    return out, hout


def kernel(x, hidden_states, masks, W_ih_0, W_hh_0, b_ih_0, b_hh_0,
           W_ih_1, W_hh_1, b_ih_1, b_hh_1, *, interpret=False):
    mf = masks.astype(jnp.float32)
    b0 = (b_ih_0 + b_hh_0).reshape(1, G)
    b1 = (b_ih_1 + b_hh_1).reshape(1, G)
    out, hout = _run(x, hidden_states, mf, W_ih_0, W_hh_0, W_ih_1, W_hh_1,
                     b0, b1, interpret=interpret)
    return out, hout


# speedup vs baseline: 4.0783x; 1.1803x over previous
import functools

import jax
import jax.numpy as jnp
from jax.experimental import pallas as pl
from jax.experimental.pallas import tpu as pltpu

N = 4096
H = 512
G = 4 * H
BN = 512
SPLIT = 2


def _sigmoid(x):
    return 0.5 * (jnp.tanh(0.5 * x) + 1.0)


def _dot_t(a, b):
    return jax.lax.dot_general(
        a, b, dimension_numbers=(((1,), (1,)), ((), ())),
        preferred_element_type=jnp.float32)


def _hid_in_copy(hid_hbm, hin_buf, in_sems, step, slot, j):
    return pltpu.make_async_copy(
        hid_hbm.at[pl.ds(step * BN, BN), j],
        hin_buf.at[slot, j],
        in_sems.at[slot, j])


def _hid_out_copy(hout_hbm, hout_buf, out_sems, step, slot, j):
    return pltpu.make_async_copy(
        hout_buf.at[slot, j],
        hout_hbm.at[pl.ds(step * BN, BN), j],
        out_sems.at[slot, j])


def _cell_chain(x_ref, m_ref, hin_buf, slot, wi0, wh0, wi1, wh1, b0, b1,
                out_ref, hout_buf, lo, rows):
    sub = pl.ds(lo, rows)
    m = m_ref[sub, :]
    h0 = hin_buf[slot, 0, sub, :] * m
    h1 = hin_buf[slot, 1, sub, :] * m
    c0 = hin_buf[slot, 2, sub, :] * m
    c1 = hin_buf[slot, 3, sub, :] * m

    xb = x_ref[sub, :].astype(jnp.bfloat16)
    gates0 = _dot_t(xb, wi0) + _dot_t(h0.astype(jnp.bfloat16), wh0) + b0
    i0 = _sigmoid(gates0[:, 0 * H:1 * H])
    f0 = _sigmoid(gates0[:, 1 * H:2 * H])
    g0 = jnp.tanh(gates0[:, 2 * H:3 * H])
    o0 = _sigmoid(gates0[:, 3 * H:4 * H])
    c0n = f0 * c0 + i0 * g0
    h0n = o0 * jnp.tanh(c0n)

    gates1 = (_dot_t(h0n.astype(jnp.bfloat16), wi1)
              + _dot_t(h1.astype(jnp.bfloat16), wh1) + b1)
    i1 = _sigmoid(gates1[:, 0 * H:1 * H])
    f1 = _sigmoid(gates1[:, 1 * H:2 * H])
    g1 = jnp.tanh(gates1[:, 2 * H:3 * H])
    o1 = _sigmoid(gates1[:, 3 * H:4 * H])
    c1n = f1 * c1 + i1 * g1
    h1n = o1 * jnp.tanh(c1n)

    out_ref[sub, :] = h1n
    hout_buf[slot, 0, sub, :] = h0n
    hout_buf[slot, 1, sub, :] = h1n
    hout_buf[slot, 2, sub, :] = c0n
    hout_buf[slot, 3, sub, :] = c1n


def _lstm_kernel(x_ref, m_ref, wi0_ref, wh0_ref, wi1_ref, wh1_ref,
                 bi0_ref, bh0_ref, bi1_ref, bh1_ref, hid_hbm,
                 out_ref, hout_hbm, hin_buf, hout_buf, wbuf, in_sems, out_sems):
    i = pl.program_id(0)
    nsteps = pl.num_programs(0)
    slot = jax.lax.rem(i, 2)
    nslot = jax.lax.rem(i + 1, 2)

    @pl.when(i == 0)
    def _():
        for j in range(4):
            _hid_in_copy(hid_hbm, hin_buf, in_sems, 0, 0, j).start()

    @pl.when(i + 1 < nsteps)
    def _():
        for j in range(4):
            _hid_in_copy(hid_hbm, hin_buf, in_sems, i + 1, nslot, j).start()

    for j in range(4):
        _hid_in_copy(hid_hbm, hin_buf, in_sems, i, slot, j).wait()

    @pl.when(i >= 2)
    def _():
        for j in range(4):
            _hid_out_copy(hout_hbm, hout_buf, out_sems, i - 2, slot, j).wait()

    @pl.when(i == 0)
    def _():
        wbuf[0] = wi0_ref[...].astype(jnp.bfloat16)
        wbuf[1] = wh0_ref[...].astype(jnp.bfloat16)
        wbuf[2] = wi1_ref[...].astype(jnp.bfloat16)
        wbuf[3] = wh1_ref[...].astype(jnp.bfloat16)

    wi0 = wbuf[0]
    wh0 = wbuf[1]
    wi1 = wbuf[2]
    wh1 = wbuf[3]
    b0 = bi0_ref[...] + bh0_ref[...]
    b1 = bi1_ref[...] + bh1_ref[...]

    rows = BN // SPLIT
    for s in range(SPLIT):
        _cell_chain(x_ref, m_ref, hin_buf, slot, wi0, wh0, wi1, wh1, b0, b1,
                    out_ref, hout_buf, s * rows, rows)

    for j in range(4):
        _hid_out_copy(hout_hbm, hout_buf, out_sems, i, slot, j).start()

    @pl.when(i == nsteps - 1)
    def _():
        for j in range(4):
            _hid_out_copy(hout_hbm, hout_buf, out_sems, i - 1, nslot, j).wait()
            _hid_out_copy(hout_hbm, hout_buf, out_sems, i, slot, j).wait()


@functools.partial(jax.jit, static_argnames=("interpret",))
def _run(x, hs, mf, wi0, wh0, wi1, wh1, bi0, bh0, bi1, bh1, interpret=False):
    grid = (N // BN,)
    row = lambda i: (i, 0)
    rep = lambda i: (0, 0)
    out, hout = pl.pallas_call(
        _lstm_kernel,
        grid=grid,
        in_specs=[
            pl.BlockSpec((BN, H), row),
            pl.BlockSpec((BN, 1), row),
            pl.BlockSpec((G, H), rep),
            pl.BlockSpec((G, H), rep),
            pl.BlockSpec((G, H), rep),
            pl.BlockSpec((G, H), rep),
            pl.BlockSpec((1, G), rep),
            pl.BlockSpec((1, G), rep),
            pl.BlockSpec((1, G), rep),
            pl.BlockSpec((1, G), rep),
            pl.BlockSpec(memory_space=pltpu.MemorySpace.HBM),
        ],
        out_specs=[
            pl.BlockSpec((BN, H), row),
            pl.BlockSpec(memory_space=pltpu.MemorySpace.HBM),
        ],
        out_shape=[
            jax.ShapeDtypeStruct((N, H), jnp.float32),
            jax.ShapeDtypeStruct((N, 4, H), jnp.float32),
        ],
        scratch_shapes=[
            pltpu.VMEM((2, 4, BN, H), jnp.float32),
            pltpu.VMEM((2, 4, BN, H), jnp.float32),
            pltpu.VMEM((4, G, H), jnp.bfloat16),
            pltpu.SemaphoreType.DMA((2, 4)),
            pltpu.SemaphoreType.DMA((2, 4)),
        ],
        compiler_params=pltpu.CompilerParams(
            dimension_semantics=("arbitrary",),
        ),
        interpret=interpret,
    )(x, mf, wi0, wh0, wi1, wh1, bi0, bh0, bi1, bh1, hs)
    return out, hout


def kernel(x, hidden_states, masks, W_ih_0, W_hh_0, b_ih_0, b_hh_0,
           W_ih_1, W_hh_1, b_ih_1, b_hh_1, *, interpret=False):
    mf = masks.astype(jnp.float32)
    out, hout = _run(x, hidden_states, mf, W_ih_0, W_hh_0, W_ih_1, W_hh_1,
                     b_ih_0.reshape(1, G), b_hh_0.reshape(1, G),
                     b_ih_1.reshape(1, G), b_hh_1.reshape(1, G),
                     interpret=interpret)
    return out, hout


# speedup vs baseline: 4.1284x; 1.0123x over previous
import functools

import jax
import jax.numpy as jnp
from jax.experimental import pallas as pl
from jax.experimental.pallas import tpu as pltpu

N = 4096
H = 512
G = 4 * H
BN = 512
SPLIT = 2


def _sigmoid(x):
    return 0.5 * (jnp.tanh(0.5 * x) + 1.0)


def _dot_t(a, b):
    return jax.lax.dot_general(
        a, b, dimension_numbers=(((1,), (1,)), ((), ())),
        preferred_element_type=jnp.float32)


def _hid_in_copy(hid_hbm, hin_buf, in_sems, step, slot, j):
    return pltpu.make_async_copy(
        hid_hbm.at[pl.ds(step * BN, BN), j],
        hin_buf.at[slot, j],
        in_sems.at[slot, j])


def _hid_out_copy(hout_hbm, hout_buf, out_sems, step, slot, j):
    return pltpu.make_async_copy(
        hout_buf.at[slot, j],
        hout_hbm.at[pl.ds(step * BN, BN), j],
        out_sems.at[slot, j])


def _out_copy(out_hbm, hout_buf, out_sems, step, slot):
    return pltpu.make_async_copy(
        hout_buf.at[slot, 1],
        out_hbm.at[pl.ds(step * BN, BN)],
        out_sems.at[slot, 4])


def _cell_chain(x_ref, m_ref, hin_buf, slot, wi0, wh0, wi1, wh1, b0, b1,
                hout_buf, lo, rows):
    sub = pl.ds(lo, rows)
    m = m_ref[sub, :]
    h0 = hin_buf[slot, 0, sub, :] * m
    h1 = hin_buf[slot, 1, sub, :] * m
    c0 = hin_buf[slot, 2, sub, :] * m
    c1 = hin_buf[slot, 3, sub, :] * m

    xb = x_ref[sub, :].astype(jnp.bfloat16)
    gates0 = _dot_t(xb, wi0) + _dot_t(h0.astype(jnp.bfloat16), wh0) + b0
    i0 = _sigmoid(gates0[:, 0 * H:1 * H])
    f0 = _sigmoid(gates0[:, 1 * H:2 * H])
    g0 = jnp.tanh(gates0[:, 2 * H:3 * H])
    o0 = _sigmoid(gates0[:, 3 * H:4 * H])
    c0n = f0 * c0 + i0 * g0
    h0n = o0 * jnp.tanh(c0n)

    gates1 = (_dot_t(h0n.astype(jnp.bfloat16), wi1)
              + _dot_t(h1.astype(jnp.bfloat16), wh1) + b1)
    i1 = _sigmoid(gates1[:, 0 * H:1 * H])
    f1 = _sigmoid(gates1[:, 1 * H:2 * H])
    g1 = jnp.tanh(gates1[:, 2 * H:3 * H])
    o1 = _sigmoid(gates1[:, 3 * H:4 * H])
    c1n = f1 * c1 + i1 * g1
    h1n = o1 * jnp.tanh(c1n)

    hout_buf[slot, 0, sub, :] = h0n
    hout_buf[slot, 1, sub, :] = h1n
    hout_buf[slot, 2, sub, :] = c0n
    hout_buf[slot, 3, sub, :] = c1n


def _lstm_kernel(x_ref, m_ref, wi0_ref, wh0_ref, wi1_ref, wh1_ref,
                 bi0_ref, bh0_ref, bi1_ref, bh1_ref, hid_hbm,
                 out_ref, hout_hbm, hin_buf, hout_buf, wbuf, in_sems, out_sems):
    i = pl.program_id(0)
    nsteps = pl.num_programs(0)
    slot = jax.lax.rem(i, 2)
    nslot = jax.lax.rem(i + 1, 2)

    @pl.when(i == 0)
    def _():
        for j in range(4):
            _hid_in_copy(hid_hbm, hin_buf, in_sems, 0, 0, j).start()

    @pl.when(i + 1 < nsteps)
    def _():
        for j in range(4):
            _hid_in_copy(hid_hbm, hin_buf, in_sems, i + 1, nslot, j).start()

    for j in range(4):
        _hid_in_copy(hid_hbm, hin_buf, in_sems, i, slot, j).wait()

    @pl.when(i >= 2)
    def _():
        for j in range(4):
            _hid_out_copy(hout_hbm, hout_buf, out_sems, i - 2, slot, j).wait()
        _out_copy(out_ref, hout_buf, out_sems, i - 2, slot).wait()

    @pl.when(i == 0)
    def _():
        wbuf[0] = wi0_ref[...].astype(jnp.bfloat16)
        wbuf[1] = wh0_ref[...].astype(jnp.bfloat16)
        wbuf[2] = wi1_ref[...].astype(jnp.bfloat16)
        wbuf[3] = wh1_ref[...].astype(jnp.bfloat16)

    wi0 = wbuf[0]
    wh0 = wbuf[1]
    wi1 = wbuf[2]
    wh1 = wbuf[3]
    b0 = bi0_ref[...] + bh0_ref[...]
    b1 = bi1_ref[...] + bh1_ref[...]

    rows = BN // SPLIT
    for s in range(SPLIT):
        _cell_chain(x_ref, m_ref, hin_buf, slot, wi0, wh0, wi1, wh1, b0, b1,
                    hout_buf, s * rows, rows)

    for j in range(4):
        _hid_out_copy(hout_hbm, hout_buf, out_sems, i, slot, j).start()
    _out_copy(out_ref, hout_buf, out_sems, i, slot).start()

    @pl.when(i == nsteps - 1)
    def _():
        for j in range(4):
            _hid_out_copy(hout_hbm, hout_buf, out_sems, i - 1, nslot, j).wait()
            _hid_out_copy(hout_hbm, hout_buf, out_sems, i, slot, j).wait()
        _out_copy(out_ref, hout_buf, out_sems, i - 1, nslot).wait()
        _out_copy(out_ref, hout_buf, out_sems, i, slot).wait()


@functools.partial(jax.jit, static_argnames=("interpret",))
def _run(x, hs, mf, wi0, wh0, wi1, wh1, bi0, bh0, bi1, bh1, interpret=False):
    grid = (N // BN,)
    row = lambda i: (i, 0)
    rep = lambda i: (0, 0)
    out, hout = pl.pallas_call(
        _lstm_kernel,
        grid=grid,
        in_specs=[
            pl.BlockSpec((BN, H), row),
            pl.BlockSpec((BN, 1), row),
            pl.BlockSpec((G, H), rep),
            pl.BlockSpec((G, H), rep),
            pl.BlockSpec((G, H), rep),
            pl.BlockSpec((G, H), rep),
            pl.BlockSpec((1, G), rep),
            pl.BlockSpec((1, G), rep),
            pl.BlockSpec((1, G), rep),
            pl.BlockSpec((1, G), rep),
            pl.BlockSpec(memory_space=pltpu.MemorySpace.HBM),
        ],
        out_specs=[
            pl.BlockSpec(memory_space=pltpu.MemorySpace.HBM),
            pl.BlockSpec(memory_space=pltpu.MemorySpace.HBM),
        ],
        out_shape=[
            jax.ShapeDtypeStruct((N, H), jnp.float32),
            jax.ShapeDtypeStruct((N, 4, H), jnp.float32),
        ],
        scratch_shapes=[
            pltpu.VMEM((2, 4, BN, H), jnp.float32),
            pltpu.VMEM((2, 4, BN, H), jnp.float32),
            pltpu.VMEM((4, G, H), jnp.bfloat16),
            pltpu.SemaphoreType.DMA((2, 4)),
            pltpu.SemaphoreType.DMA((2, 5)),
        ],
        compiler_params=pltpu.CompilerParams(
            dimension_semantics=("arbitrary",),
        ),
        interpret=interpret,
    )(x, mf, wi0, wh0, wi1, wh1, bi0, bh0, bi1, bh1, hs)
    return out, hout


def kernel(x, hidden_states, masks, W_ih_0, W_hh_0, b_ih_0, b_hh_0,
           W_ih_1, W_hh_1, b_ih_1, b_hh_1, *, interpret=False):
    mf = masks.astype(jnp.float32)
    out, hout = _run(x, hidden_states, mf, W_ih_0, W_hh_0, W_ih_1, W_hh_1,
                     b_ih_0.reshape(1, G), b_hh_0.reshape(1, G),
                     b_ih_1.reshape(1, G), b_hh_1.reshape(1, G),
                     interpret=interpret)
    return out, hout
